# Initial kernel scaffold; baseline (speedup 1.0000x reference)
import jax
import jax.numpy as jnp
from jax.experimental import pallas as pl


def kernel(x, dw_w, dw_b, ln_w, ln_b, w1, b1, w2, b2, gamma):
    raise NotImplementedError("write your pallas kernel here")


# speedup vs baseline: 10.2639x; 10.2639x over previous
import functools
import math

import jax
import jax.numpy as jnp
from jax.experimental import pallas as pl
from jax.experimental.pallas import tpu as pltpu

_INV_SQRT2 = 1.0 / math.sqrt(2.0)


def _block_kernel(x_ref, wtap_ref, dwb_ref, lnw_ref, lnb_ref, w1_ref, b1_ref,
                  w2_ref, b2_ref, g_ref, o_ref, xpad_ref, *, H, W, C, K, eps):
    P = K // 2
    xpad_ref[...] = jnp.zeros_like(xpad_ref)
    x = x_ref[...].astype(jnp.float32)
    xpad_ref[P:P + H, P:P + W, :] = x
    acc = jnp.zeros((H, W, C), jnp.float32)
    for ky in range(K):
        for kx in range(K):
            tap = wtap_ref[ky * K + kx, :].reshape(1, 1, C)
            acc = acc + xpad_ref[ky:ky + H, kx:kx + W, :] * tap
    dw = acc.reshape(H * W, C) + dwb_ref[...]
    mean = jnp.mean(dw, axis=-1, keepdims=True)
    mean_sq = jnp.mean(dw * dw, axis=-1, keepdims=True)
    var = mean_sq - mean * mean
    y = (dw - mean) * jax.lax.rsqrt(var + eps)
    y = y * lnw_ref[...] + lnb_ref[...]
    h = jnp.dot(y, w1_ref[...], preferred_element_type=jnp.float32)
    h = h + b1_ref[...]
    h = 0.5 * h * (1.0 + jax.lax.erf(h * _INV_SQRT2))
    z = jnp.dot(h, w2_ref[...], preferred_element_type=jnp.float32)
    z = z + b2_ref[...]
    out = x.reshape(H * W, C) + z * g_ref[...]
    o_ref[...] = out.reshape(H, W, C).astype(o_ref.dtype)


def kernel(x, dw_w, dw_b, ln_w, ln_b, w1, b1, w2, b2, gamma):
    N, C, H, W = x.shape
    K = 7
    P = K // 2
    H4 = w1.shape[0]
    eps = 1e-6

    x_nhwc = jnp.transpose(x, (0, 2, 3, 1))
    wtap = dw_w.reshape(C, K * K).T.astype(jnp.float32)
    KK = ((K * K + 7) // 8) * 8
    wtap = jnp.pad(wtap, ((0, KK - K * K), (0, 0)))

    def fullspec(shape):
        return pl.BlockSpec(shape, lambda n: (0,) * len(shape))

    y_nhwc = pl.pallas_call(
        functools.partial(_block_kernel, H=H, W=W, C=C, K=K, eps=eps),
        out_shape=jax.ShapeDtypeStruct((N, H, W, C), x.dtype),
        grid=(N,),
        in_specs=[
            pl.BlockSpec((None, H, W, C), lambda n: (n, 0, 0, 0)),
            fullspec((KK, C)),
            fullspec((1, C)),
            fullspec((1, C)),
            fullspec((1, C)),
            fullspec((C, H4)),
            fullspec((1, H4)),
            fullspec((H4, C)),
            fullspec((1, C)),
            fullspec((1, C)),
        ],
        out_specs=pl.BlockSpec((None, H, W, C), lambda n: (n, 0, 0, 0)),
        scratch_shapes=[pltpu.VMEM((H + 2 * P, W + 2 * P, C), jnp.float32)],
        compiler_params=pltpu.CompilerParams(
            dimension_semantics=("parallel",),
            vmem_limit_bytes=48 * 1024 * 1024),
    )(x_nhwc,
      wtap,
      dw_b.reshape(1, C).astype(jnp.float32),
      ln_w.reshape(1, C).astype(jnp.float32),
      ln_b.reshape(1, C).astype(jnp.float32),
      w1.T.astype(jnp.float32),
      b1.reshape(1, H4).astype(jnp.float32),
      w2.T.astype(jnp.float32),
      b2.reshape(1, C).astype(jnp.float32),
      gamma.reshape(1, C).astype(jnp.float32))
    return jnp.transpose(y_nhwc, (0, 3, 1, 2))
